# baseline (speedup 1.0000x reference)
import functools

import jax
import jax.numpy as jnp
from jax import lax
from jax.experimental import pallas as pl
from jax.experimental.pallas import tpu as pltpu
from jax.experimental.pallas import tpu_sc as plsc

N = 16384
M = 64
K = 8
NUM_CORES = 2
NUM_SUBCORES = 16
NW = NUM_CORES * NUM_SUBCORES
RW = N // NW
L = 16
CH = 128
NCH = RW // CH
GROUPS = CH // L
TS = 17


def _qb_gating_body(logits_hbm, beta_hbm, out_hbm, in_v, out_v, beta_v, st_v):
    wid = lax.axis_index("s") * NUM_CORES + lax.axis_index("c")
    rbase = wid * RW

    pltpu.sync_copy(beta_hbm, beta_v)

    iota16 = lax.iota(jnp.int32, 16)
    zeros16 = jnp.zeros((L,), jnp.float32)
    beta_regs = [beta_v[pl.ds(c * L, L)] for c in range(M // L)]
    taddr = [(c * L + iota16) * TS for c in range(M // L)]
    lowb = [63 - (c * L + iota16) for c in range(M // L)]


    S8 = [(0, 1), (2, 3), (4, 5), (6, 7),
          (0, 2), (1, 3), (4, 6), (5, 7),
          (1, 2), (5, 6),
          (0, 4), (1, 5), (2, 6), (3, 7),
          (2, 4), (3, 5),
          (1, 2), (3, 4), (5, 6)]
    C8 = [(0, 4), (1, 5), (2, 6), (3, 7),
          (0, 2), (1, 3), (4, 6), (5, 7),
          (0, 1), (2, 3), (4, 5), (6, 7)]

    def sort8(v):
        for i, j in S8:
            hi = jnp.maximum(v[i], v[j])
            lo = jnp.minimum(v[i], v[j])
            v[i], v[j] = hi, lo
        return v

    def merge_top8(a, b):
        t = [jnp.maximum(a[i], b[7 - i]) for i in range(K)]
        for i, j in C8:
            hi = jnp.maximum(t[i], t[j])
            lo = jnp.minimum(t[i], t[j])
            t[i], t[j] = hi, lo
        return t

    def one_group(g, sbase):
        rows = g * L + iota16
        grow = g * L

        for r in range(L):
            for c in range(M // L):
                out_v[grow + r, pl.ds(c * L, L)] = zeros16

        for r in range(L):
            for c in range(M // L):
                v = in_v[grow + r, pl.ds(c * L, L)] - beta_regs[c]
                b = plsc.bitcast(v, jnp.int32)
                sr = lax.shift_right_arithmetic(b, 31)
                key = (b & ~63) | (lowb[c] ^ (sr & 63))
                plsc.store_scatter(st_v, [sbase + taddr[c] + r], key)

        acc = sort8([plsc.bitcast(st_v[pl.ds(sbase + j * TS, L)], jnp.float32)
                     for j in range(K)])
        for cb in range(1, M // K):
            nxt = sort8(
                [plsc.bitcast(
                    st_v[pl.ds(sbase + (cb * K + j) * TS, L)], jnp.float32)
                 for j in range(K)]
            )
            acc = merge_top8(acc, nxt)

        idxs = []
        for k in range(K):
            ki = plsc.bitcast(acc[k], jnp.int32)
            sr = lax.shift_right_arithmetic(ki, 31)
            idxs.append(63 - ((ki & 63) ^ (sr & 63)))

        rk = [plsc.load_gather(in_v, [rows, idxs[k]]) for k in range(K)]
        mx = rk[0]
        for k in range(1, K):
            mx = jnp.maximum(mx, rk[k])
        ek = [jnp.exp(rk[k] - mx) for k in range(K)]
        den = ek[0]
        for k in range(1, K):
            den = den + ek[k]
        recip = jnp.float32(1.0) / den
        for k in range(K):
            plsc.store_scatter(out_v, [rows, idxs[k]], ek[k] * recip)

    for ch in range(NCH):
        pltpu.sync_copy(logits_hbm.at[pl.ds(rbase + ch * CH, CH)], in_v)

        @plsc.parallel_loop(0, GROUPS, unroll=2)
        def _loop(g):
            one_group(g, g * (M * TS))

        pltpu.sync_copy(out_v, out_hbm.at[pl.ds(rbase + ch * CH, CH)])


def _qb_gating(logits2d, beta_qb):
    mesh = plsc.VectorSubcoreMesh(core_axis_name="c", subcore_axis_name="s")
    run = functools.partial(
        pl.kernel,
        mesh=mesh,
        out_type=jax.ShapeDtypeStruct((N, M), jnp.float32),
        scratch_types=[
            pltpu.VMEM((CH, M), jnp.float32),
            pltpu.VMEM((CH, M), jnp.float32),
            pltpu.VMEM((M,), jnp.float32),
            pltpu.VMEM((GROUPS * M * TS,), jnp.int32),
        ],
        compiler_params=pltpu.CompilerParams(
            needs_layout_passes=False, use_tc_tiling_on_sc=True
        ),
    )(_qb_gating_body)
    return run(logits2d, beta_qb)


def kernel(logits, beta_qb):
    orig_shape = logits.shape
    flat = logits.reshape(-1, orig_shape[-1])
    assert flat.shape == (N, M), flat.shape
    probs = _qb_gating(flat, beta_qb)
    return probs.reshape(orig_shape)


# speedup vs baseline: 1.0971x; 1.0971x over previous
import functools

import jax
import jax.numpy as jnp
from jax import lax
from jax.experimental import pallas as pl
from jax.experimental.pallas import tpu as pltpu
from jax.experimental.pallas import tpu_sc as plsc

N = 16384
M = 64
K = 8
NUM_CORES = 2
NUM_SUBCORES = 16
NW = NUM_CORES * NUM_SUBCORES
RW = N // NW
L = 16
CH = 128
NCH = RW // CH
GROUPS = CH // L
TS = 17


def _qb_gating_body(
    logits_hbm, beta_hbm, out_hbm,
    in_v0, in_v1, out_v0, out_v1, beta_v, st_v,
    sem_i0, sem_i1, sem_o0, sem_o1,
):
    wid = lax.axis_index("s") * NUM_CORES + lax.axis_index("c")
    rbase = wid * RW
    in_bufs = (in_v0, in_v1)
    out_bufs = (out_v0, out_v1)
    in_sems = (sem_i0, sem_i1)
    out_sems = (sem_o0, sem_o1)

    pltpu.sync_copy(beta_hbm, beta_v)

    iota16 = lax.iota(jnp.int32, 16)
    zeros16 = jnp.zeros((L,), jnp.float32)
    beta_regs = [beta_v[pl.ds(c * L, L)] for c in range(M // L)]
    taddr = [(c * L + iota16) * TS for c in range(M // L)]
    lowb = [63 - (c * L + iota16) for c in range(M // L)]


    S8 = [(0, 1), (2, 3), (4, 5), (6, 7),
          (0, 2), (1, 3), (4, 6), (5, 7),
          (1, 2), (5, 6),
          (0, 4), (1, 5), (2, 6), (3, 7),
          (2, 4), (3, 5),
          (1, 2), (3, 4), (5, 6)]
    C8 = [(0, 4), (1, 5), (2, 6), (3, 7),
          (0, 2), (1, 3), (4, 6), (5, 7),
          (0, 1), (2, 3), (4, 5), (6, 7)]

    def sort8(v):
        for i, j in S8:
            hi = jnp.maximum(v[i], v[j])
            lo = jnp.minimum(v[i], v[j])
            v[i], v[j] = hi, lo
        return v

    def merge_top8(a, b):
        t = [jnp.maximum(a[i], b[7 - i]) for i in range(K)]
        for i, j in C8:
            hi = jnp.maximum(t[i], t[j])
            lo = jnp.minimum(t[i], t[j])
            t[i], t[j] = hi, lo
        return t

    def one_group(g, sbase, in_v, out_v):
        rows = g * L + iota16
        grow = g * L

        for r in range(L):
            for c in range(M // L):
                out_v[grow + r, pl.ds(c * L, L)] = zeros16

        for r in range(L):
            for c in range(M // L):
                v = in_v[grow + r, pl.ds(c * L, L)] - beta_regs[c]
                b = plsc.bitcast(v, jnp.int32)
                sr = lax.shift_right_arithmetic(b, 31)
                key = (b & ~63) | (lowb[c] ^ (sr & 63))
                plsc.store_scatter(st_v, [sbase + taddr[c] + r], key)

        acc = sort8([plsc.bitcast(st_v[pl.ds(sbase + j * TS, L)], jnp.float32)
                     for j in range(K)])
        for cb in range(1, M // K):
            nxt = sort8(
                [plsc.bitcast(
                    st_v[pl.ds(sbase + (cb * K + j) * TS, L)], jnp.float32)
                 for j in range(K)]
            )
            acc = merge_top8(acc, nxt)

        idxs = []
        for k in range(K):
            ki = plsc.bitcast(acc[k], jnp.int32)
            sr = lax.shift_right_arithmetic(ki, 31)
            idxs.append(63 - ((ki & 63) ^ (sr & 63)))

        rk = [plsc.load_gather(in_v, [rows, idxs[k]]) for k in range(K)]
        mx = rk[0]
        for k in range(1, K):
            mx = jnp.maximum(mx, rk[k])
        ek = [jnp.exp(rk[k] - mx) for k in range(K)]
        den = ek[0]
        for k in range(1, K):
            den = den + ek[k]
        recip = jnp.float32(1.0) / den
        for k in range(K):
            plsc.store_scatter(out_v, [rows, idxs[k]], ek[k] * recip)

    h_in = [None, None]
    h_out = [None, None]
    h_in[0] = pltpu.async_copy(
        logits_hbm.at[pl.ds(rbase, CH)], in_bufs[0], in_sems[0]
    )
    for ch in range(NCH):
        b = ch % 2
        if ch + 1 < NCH:
            h_in[1 - b] = pltpu.async_copy(
                logits_hbm.at[pl.ds(rbase + (ch + 1) * CH, CH)],
                in_bufs[1 - b],
                in_sems[1 - b],
            )
        h_in[b].wait()
        if h_out[b] is not None:
            h_out[b].wait()

        @plsc.parallel_loop(0, GROUPS, unroll=2)
        def _loop(g, _b=b):
            one_group(g, (_b * GROUPS + g) * (M * TS), in_bufs[_b], out_bufs[_b])

        h_out[b] = pltpu.async_copy(
            out_bufs[b], out_hbm.at[pl.ds(rbase + ch * CH, CH)], out_sems[b]
        )
    h_out[0].wait()
    h_out[1].wait()


def _qb_gating(logits2d, beta_qb):
    mesh = plsc.VectorSubcoreMesh(core_axis_name="c", subcore_axis_name="s")
    run = functools.partial(
        pl.kernel,
        mesh=mesh,
        out_type=jax.ShapeDtypeStruct((N, M), jnp.float32),
        scratch_types=[
            pltpu.VMEM((CH, M), jnp.float32),
            pltpu.VMEM((CH, M), jnp.float32),
            pltpu.VMEM((CH, M), jnp.float32),
            pltpu.VMEM((CH, M), jnp.float32),
            pltpu.VMEM((M,), jnp.float32),
            pltpu.VMEM((2 * GROUPS * M * TS,), jnp.int32),
            pltpu.SemaphoreType.DMA,
            pltpu.SemaphoreType.DMA,
            pltpu.SemaphoreType.DMA,
            pltpu.SemaphoreType.DMA,
        ],
        compiler_params=pltpu.CompilerParams(
            needs_layout_passes=False, use_tc_tiling_on_sc=True
        ),
    )(_qb_gating_body)
    return run(logits2d, beta_qb)


def kernel(logits, beta_qb):
    orig_shape = logits.shape
    flat = logits.reshape(-1, orig_shape[-1])
    assert flat.shape == (N, M), flat.shape
    probs = _qb_gating(flat, beta_qb)
    return probs.reshape(orig_shape)
